# Initial kernel scaffold; baseline (speedup 1.0000x reference)
import functools
import math

import jax
import jax.numpy as jnp
from jax import lax
from jax.experimental import pallas as pl
from jax.experimental.pallas import tpu as pltpu
from jax.experimental.pallas import tpu_sc as plsc

D_MODEL = 128
SCALE = math.sqrt(float(D_MODEL))
CHUNK = 128


def _sc_embed_concat(xf, xi, lut, *, n_rows, n_feat):
    f_keep = n_feat - 1
    d_out = f_keep + D_MODEL

    info = plsc.get_sparse_core_info()
    nc, ns = info.num_cores, info.num_subcores
    nw = nc * ns
    rows_per_w = n_rows // nw
    n_chunks = rows_per_w // CHUNK

    mesh = plsc.VectorSubcoreMesh(core_axis_name="c", subcore_axis_name="s")

    @functools.partial(
        pl.kernel,
        mesh=mesh,
        out_type=jax.ShapeDtypeStruct((n_rows, d_out), jnp.float32),
        scratch_types=[
            pltpu.VMEM((CHUNK,), jnp.int32),
            pltpu.VMEM((CHUNK, d_out), jnp.float32),
            pltpu.SemaphoreType.DMA,
        ],
    )
    def k(x_hbm, xi_hbm, lut_hbm, out_hbm, idx_v, out_v, sem):
        wid = lax.axis_index("s") * nc + lax.axis_index("c")
        w_base = wid * rows_per_w

        def chunk_body(ci, carry):
            base = w_base + ci * CHUNK
            pltpu.sync_copy(xi_hbm.at[pl.ds(base, CHUNK)], idx_v)
            gather = pltpu.async_copy(
                lut_hbm.at[idx_v], out_v.at[:, pl.ds(f_keep, D_MODEL)], sem
            )
            pltpu.sync_copy(
                x_hbm.at[pl.ds(base, CHUNK), pl.ds(0, f_keep)],
                out_v.at[:, pl.ds(0, f_keep)],
            )
            gather.wait()

            def row_body(i, c2):
                for j in range(D_MODEL // 16):
                    sl = pl.ds(f_keep + j * 16, 16)
                    out_v[i, sl] = out_v[i, sl] * SCALE
                return c2

            lax.fori_loop(0, CHUNK, row_body, 0)
            pltpu.sync_copy(out_v, out_hbm.at[pl.ds(base, CHUNK), :])
            return carry

        lax.fori_loop(0, n_chunks, chunk_body, 0)

    return k(xf, xi, lut)


def kernel(x, lut):
    b, s, f = x.shape
    n = b * s
    xf = x.reshape(n, f)
    xi = xf[:, f - 1].astype(jnp.int32)
    out = _sc_embed_concat(xf, xi, lut, n_rows=n, n_feat=f)
    return out.reshape(b, s, f - 1 + D_MODEL)


# speedup vs baseline: 1.0278x; 1.0278x over previous
import functools
import math

import jax
import jax.numpy as jnp
from jax import lax
from jax.experimental import pallas as pl
from jax.experimental.pallas import tpu as pltpu
from jax.experimental.pallas import tpu_sc as plsc

D_MODEL = 128
SCALE = math.sqrt(float(D_MODEL))
CHUNK = 128


def _sc_embed_concat(xflat, xi, lut, *, n_rows, n_feat):
    f_keep = n_feat - 1
    d_out = f_keep + D_MODEL

    info = plsc.get_sparse_core_info()
    nc, ns = info.num_cores, info.num_subcores
    nw = nc * ns
    rows_per_w = n_rows // nw
    n_chunks = rows_per_w // CHUNK

    mesh = plsc.VectorSubcoreMesh(core_axis_name="c", subcore_axis_name="s")

    @functools.partial(
        pl.kernel,
        mesh=mesh,
        out_type=jax.ShapeDtypeStruct((n_rows * d_out,), jnp.float32),
        scratch_types=[
            pltpu.VMEM((CHUNK,), jnp.int32),
            pltpu.VMEM((CHUNK, D_MODEL), jnp.float32),
            pltpu.VMEM((CHUNK * n_feat,), jnp.float32),
            pltpu.VMEM((CHUNK * d_out,), jnp.float32),
            pltpu.SemaphoreType.DMA,
        ],
    )
    def k(x_hbm, xi_hbm, lut_hbm, out_hbm, idx_v, emb_v, x_v, out_v, sem):
        wid = lax.axis_index("s") * nc + lax.axis_index("c")
        w_base = wid * rows_per_w

        def chunk_body(ci, carry):
            base = w_base + ci * CHUNK
            pltpu.sync_copy(xi_hbm.at[pl.ds(base, CHUNK)], idx_v)
            gather = pltpu.async_copy(lut_hbm.at[idx_v], emb_v, sem)
            pltpu.sync_copy(
                x_hbm.at[pl.ds(base * n_feat, CHUNK * n_feat)], x_v
            )
            gather.wait()

            def row_body(i, c2):
                out_v[pl.ds(i * d_out, f_keep)] = x_v[pl.ds(i * n_feat, f_keep)]
                for j in range(D_MODEL // 16):
                    out_v[pl.ds(i * d_out + f_keep + j * 16, 16)] = (
                        emb_v[i, pl.ds(j * 16, 16)] * SCALE
                    )
                return c2

            lax.fori_loop(0, CHUNK, row_body, 0)
            pltpu.sync_copy(out_v, out_hbm.at[pl.ds(base * d_out, CHUNK * d_out)])
            return carry

        lax.fori_loop(0, n_chunks, chunk_body, 0)

    return k(xflat, xi, lut)


def kernel(x, lut):
    b, s, f = x.shape
    n = b * s
    xf = x.reshape(n, f)
    xi = xf[:, f - 1].astype(jnp.int32)
    out = _sc_embed_concat(x.reshape(-1), xi, lut, n_rows=n, n_feat=f)
    return out.reshape(b, s, f - 1 + D_MODEL)
